# baseline (speedup 1.0000x reference)
import functools
import math

import jax
import jax.numpy as jnp
from jax.experimental import pallas as pl

DIM = 256
HEADS = 4
TOPK = 40
MLP = 2
EPS = 1e-5
RS = 2

HD = DIM // HEADS
NEG = -1e30


def _conv1d(x, w, b=None, padding=0, groups=1):
    out = jax.lax.conv_general_dilated(
        x, w, (1,), [(padding, padding)],
        dimension_numbers=('NCH', 'OIH', 'NCH'),
        feature_group_count=groups)
    if b is not None:
        out = out + b[None, :, None]
    return out


def _bn(x, g, b):
    return x / jnp.sqrt(1.0 + EPS) * g[None, :, None] + b[None, :, None]


def _attn_kernel(q_ref, k_ref, v_ref, idx_ref, o_ref, *, nr, rb, scale):
    idxb = idx_ref[0]
    cols = jax.lax.broadcasted_iota(jnp.int32, (rb, nr), 1)
    mask = jnp.zeros((rb, nr), jnp.bool_)
    for j in range(TOPK):
        mask = jnp.logical_or(mask, cols == idxb[:, j:j + 1])

    k0 = k_ref[0, 0, 0]
    k1 = k_ref[0, 0, 1]
    v0 = v_ref[0, 0, 0]
    v1 = v_ref[0, 0, 1]
    for i in range(RS):
        q = q_ref[0, 0, i]
        s0 = jax.lax.dot_general(q, k0, (((1,), (1,)), ((), ())),
                                 preferred_element_type=jnp.float32)
        s1 = jax.lax.dot_general(q, k1, (((1,), (1,)), ((), ())),
                                 preferred_element_type=jnp.float32)
        s0 = jnp.where(mask, s0 * scale, NEG)
        s1 = jnp.where(mask, s1 * scale, NEG)
        mx = jnp.maximum(jnp.max(s0, axis=1, keepdims=True),
                         jnp.max(s1, axis=1, keepdims=True))
        p0 = jnp.exp(s0 - mx)
        p1 = jnp.exp(s1 - mx)
        den = (jnp.sum(p0, axis=1, keepdims=True)
               + jnp.sum(p1, axis=1, keepdims=True))
        o = (jax.lax.dot_general(p0, v0, (((1,), (0,)), ((), ())),
                                 preferred_element_type=jnp.float32)
             + jax.lax.dot_general(p1, v1, (((1,), (0,)), ((), ())),
                                   preferred_element_type=jnp.float32))
        o_ref[0, 0, i] = o / den


def _masked_attention(q, k, v, idx, nr):
    Bb = q.shape[0]
    rb = 256
    grid = (Bb, HEADS, nr // rb)
    scale = DIM ** (-0.5)
    kern = functools.partial(_attn_kernel, nr=nr, rb=rb, scale=scale)
    return pl.pallas_call(
        kern,
        grid=grid,
        in_specs=[
            pl.BlockSpec((1, 1, RS, rb, HD), lambda b, h, r: (b, h, 0, r, 0)),
            pl.BlockSpec((1, 1, RS, nr, HD), lambda b, h, r: (b, h, 0, 0, 0)),
            pl.BlockSpec((1, 1, RS, nr, HD), lambda b, h, r: (b, h, 0, 0, 0)),
            pl.BlockSpec((1, rb, TOPK), lambda b, h, r: (b, r, 0)),
        ],
        out_specs=pl.BlockSpec((1, 1, RS, rb, HD),
                               lambda b, h, r: (b, h, 0, r, 0)),
        out_shape=jax.ShapeDtypeStruct((Bb, HEADS, RS, nr, HD), jnp.float32),
    )(q, k, v, idx)


def _to_pos_regions(t, nr):
    Bb = t.shape[0]
    t = t.reshape(Bb, HEADS, HD, nr, RS)
    return t.transpose(0, 1, 4, 3, 2)


def _from_pos_regions(t):
    Bb = t.shape[0]
    t = t.transpose(0, 1, 4, 3, 2)
    return t.reshape(Bb, DIM, -1)


def _attention(x, p):
    Bb, C, Tt = x.shape
    nr = Tt // RS
    q = _bn(_conv1d(x, p['q_w'], padding=1), p['q_g'], p['q_b'])
    k = _bn(_conv1d(x, p['k_w'], padding=1), p['k_g'], p['k_b'])
    v = _conv1d(x, p['v_w'])

    q_r = q.reshape(Bb, C, nr, RS).mean(-1)
    k_r = k.reshape(Bb, C, nr, RS).mean(-1)
    a_r = jnp.einsum('bcr,bcs->brs', q_r, k_r)
    _, idx = jax.lax.top_k(a_r, TOPK)

    qp = _to_pos_regions(q, nr)
    kp = _to_pos_regions(k, nr)
    vp = _to_pos_regions(v, nr)
    out = _masked_attention(qp, kp, vp, idx, nr)
    out = _from_pos_regions(out)

    out = out + _conv1d(v, p['lepe_w'], p['lepe_b'], padding=1, groups=C)
    out = _conv1d(out, p['out_w'], p['out_b'])
    return out


def _mm(a, b):
    return jax.lax.dot_general(a, b, (((1,), (0,)), ((), ())),
                               preferred_element_type=jnp.float32)


def _ffn_kernel(x_ref, w1_ref, s1_ref, b1_ref, w2_ref, s2_ref, b2_ref,
                dr_ref, di_ref, rb_ref, ib_ref, c_ref, s_ref, tw_ref,
                o_ref, *, T):
    n = T // 2
    inv = 1.0 / math.sqrt(float(T))
    h = _mm(x_ref[0], w1_ref[...])
    h = jax.nn.relu(h * s1_ref[...] + b1_ref[...])
    hr = h.reshape(n, 2, h.shape[-1])
    e = hr[:, 0, :]
    o = hr[:, 1, :]
    C = c_ref[...]
    S = s_ref[...]
    er = _mm(C, e)
    ei = -_mm(S, e)
    orr = _mm(C, o)
    oi = -_mm(S, o)
    ck = tw_ref[:, 0:1]
    sk = tw_ref[:, 1:2]
    tor = ck * orr + sk * oi
    toi = ck * oi - sk * orr
    x1r = (er + tor) * inv
    x1i = (ei + toi) * inv
    x2r = (er - tor) * inv
    x2i = (ei - toi) * inv
    dr = dr_ref[...]
    di = di_ref[...]
    rb = rb_ref[...]
    ib = ib_ref[...]

    def freq_nl(xr, xi):
        yr = jax.nn.relu(xr * dr - xi * di + rb)
        yi = jax.nn.relu(xi * dr + xr * di + ib)
        return yr, yi

    x1r, x1i = freq_nl(x1r, x1i)
    x2r, x2i = freq_nl(x2r, x2i)
    u0r = x1r + x2r
    u0i = x1i + x2i
    d1r = x1r - x2r
    d1i = x1i - x2i
    v1r = d1r * ck - d1i * sk
    v1i = d1i * ck + d1r * sk
    ye = (_mm(C, u0r) - _mm(S, u0i)) * inv
    yo = (_mm(C, v1r) - _mm(S, v1i)) * inv
    o_ref[0, 0] = _mm(ye, w2_ref[...]) * s2_ref[...] + b2_ref[...]
    o_ref[0, 1] = _mm(yo, w2_ref[...]) * s2_ref[...] + b2_ref[...]


def _ffn(x, p):
    Bb, C, Tt = x.shape
    n = Tt // 2
    dh = C * MLP
    xt = x.transpose(0, 2, 1)
    w1 = p['fc1_w'][:, :, 0].T
    rs1 = 1.0 / jnp.sqrt(1.0 + EPS)
    s1 = (p['fc1_g'] * rs1)[None, :]
    b1 = p['fc1_b'][None, :]
    w2 = p['fc2_w'][:, :, 0].T
    s2 = (p['fc2_g'] * rs1)[None, :]
    b2 = p['fc2_b'][None, :]
    dr = jnp.diagonal(p['r'])[None, :]
    di = jnp.diagonal(p['i'])[None, :]
    rb = p['rb'][None, :]
    ib = p['ib'][None, :]
    kk = jnp.arange(n, dtype=jnp.float32)
    nnm = kk[:, None] * kk[None, :] * (2.0 * jnp.pi / n)
    Cm = jnp.cos(nnm)
    Sm = jnp.sin(nnm)
    ang = jnp.pi * kk / n
    tw = jnp.stack([jnp.cos(ang), jnp.sin(ang)], axis=1)

    kern = functools.partial(_ffn_kernel, T=Tt)
    full = lambda shape: pl.BlockSpec(shape, lambda b: tuple(0 for _ in shape))
    out = pl.pallas_call(
        kern,
        grid=(Bb,),
        in_specs=[
            pl.BlockSpec((1, Tt, C), lambda b: (b, 0, 0)),
            full((C, dh)), full((1, dh)), full((1, dh)),
            full((dh, C)), full((1, C)), full((1, C)),
            full((1, dh)), full((1, dh)), full((1, dh)), full((1, dh)),
            full((n, n)), full((n, n)), full((n, 2)),
        ],
        out_specs=pl.BlockSpec((1, 2, n, C), lambda b: (b, 0, 0, 0)),
        out_shape=jax.ShapeDtypeStruct((Bb, 2, n, C), jnp.float32),
    )(xt, w1, s1, b1, w2, s2, b2, dr, di, rb, ib, Cm, Sm, tw)
    return out.transpose(0, 3, 2, 1).reshape(Bb, C, Tt)


def kernel(x, params):
    x = x + _attention(_bn(x, params['n1_g'], params['n1_b']), params)
    x = x + _ffn(_bn(x, params['n2_g'], params['n2_b']), params)
    return x
